# baseline (speedup 1.0000x reference)
import functools

import jax
import jax.numpy as jnp
from jax import lax
from jax.experimental import pallas as pl
from jax.experimental.pallas import tpu as pltpu
from jax.experimental.pallas import tpu_sc as plsc

N = 10000
E = 320000
D = 128

NC = 2
NS = 16
NW = NC * NS

K = 64
CHUNKS = 160
IB = 16
NBUF = 4
EPT = K * CHUNKS
EPT_REAL = E // NW
FILL = EPT - EPT_REAL
N_PAD = 10240
ROWS_PT = N_PAD // NS

_mesh = plsc.VectorSubcoreMesh(
    core_axis_name="c", subcore_axis_name="s", num_cores=NC, num_subcores=NS)


@functools.partial(
    pl.kernel,
    out_type=jax.ShapeDtypeStruct((NC, N_PAD), jnp.float32),
    mesh=_mesh,
    scratch_types=[
        pltpu.VMEM((CHUNKS // 2, 2 * K), jnp.int32),
        pltpu.VMEM((2 * K,), jnp.float32),
        pltpu.VMEM((2 * K,), jnp.float32),
        pltpu.VMEM((ROWS_PT,), jnp.float32),
        pltpu.VMEM_SHARED((N_PAD,), jnp.float32),
    ],
)
def _sc_degree(dst_hbm, ones_hbm, zeros_hbm, deg_out, idx_v, ones_v, zeros_v,
               bounce_v, deg_sh):
    c = lax.axis_index("c")
    s = lax.axis_index("s")
    wid = s * NC + c
    row0 = s * ROWS_PT
    pltpu.sync_copy(ones_hbm, ones_v)
    pltpu.sync_copy(zeros_hbm, zeros_v)
    for j in range(ROWS_PT // (2 * K)):
        pltpu.sync_copy(zeros_v, deg_sh.at[pl.ds(row0 + j * 2 * K, 2 * K)])
    pltpu.sync_copy(dst_hbm.at[wid], idx_v)
    plsc.subcore_barrier()

    def body(j, _):
        pltpu.sync_copy(ones_v, deg_sh.at[idx_v.at[j]], add=True)
        return ()

    lax.fori_loop(0, CHUNKS // 2, body, ())
    plsc.subcore_barrier()
    pltpu.sync_copy(deg_sh.at[pl.ds(row0, ROWS_PT)], bounce_v)
    pltpu.sync_copy(bounce_v, deg_out.at[c, pl.ds(row0, ROWS_PT)])


@functools.partial(
    pl.kernel,
    out_type=jax.ShapeDtypeStruct((NC, N_PAD, D), jnp.float32),
    mesh=_mesh,
    scratch_types=[
        pltpu.VMEM((EPT,), jnp.int32),
        pltpu.VMEM((2, IB, K), jnp.int32),
        pltpu.VMEM((NBUF, K, D), jnp.float32),
        pltpu.VMEM_SHARED((N_PAD, D), jnp.float32),
        pltpu.SemaphoreType.DMA((NBUF,)),
        pltpu.SemaphoreType.DMA,
    ],
)
def _sc_scatter(hp_hbm, ei_hbm, dst_hbm, fill_hbm, zrows_hbm, out_hbm, src_v,
                dst_v, rows_v, acc_sh, sems, sem_d):
    c = lax.axis_index("c")
    s = lax.axis_index("s")
    wid = s * NC + c
    row0 = s * ROWS_PT

    def dst_block_copy(jb, bp):
        return pltpu.make_async_copy(
            dst_hbm.at[wid, pl.ds(jb * IB, IB)], dst_v.at[bp], sem_d)

    def gather(j, b):
        idx = src_v.at[pl.ds(EPT_REAL, K)]
        return pltpu.make_async_copy(
            hp_hbm.at[idx], rows_v.at[b], sems.at[b])

    pltpu.sync_copy(zrows_hbm, rows_v.at[0])
    zstores = [
        pltpu.make_async_copy(rows_v.at[0],
                              acc_sh.at[pl.ds(row0 + j * K, K)], sems.at[0])
        for j in range(ROWS_PT // K)
    ]
    for z in zstores:
        z.start()
    pltpu.sync_copy(ei_hbm.at[pl.ds(wid * EPT_REAL, EPT_REAL)],
                    src_v.at[pl.ds(0, EPT_REAL)])
    pltpu.sync_copy(fill_hbm, src_v.at[pl.ds(EPT_REAL, EPT - EPT_REAL)])
    dst_block_copy(0, 0).start()
    for z in zstores:
        z.wait()
    plsc.subcore_barrier()

    for b in range(NBUF - 1):
        gather(b, b).start()
    dst_block_copy(0, 0).wait()
    dst_block_copy(1, 1).start()

    def body(j, _):
        b = lax.rem(j, NBUF)
        jb = lax.div(j, IB)
        ji = lax.rem(j, IB)
        bp = lax.rem(jb, 2)

        @pl.when(j + NBUF - 1 < CHUNKS)
        def _():
            gather(j + NBUF - 1, lax.rem(j + NBUF - 1, NBUF)).start()

        gather(j, b).wait()

        @pl.when((ji == 0) & (jb >= 1))
        def _():
            dst_block_copy(jb, bp).wait()

            @pl.when(jb + 1 < CHUNKS // IB)
            def _():
                dst_block_copy(jb + 1, 1 - bp).start()

        pltpu.sync_copy(rows_v.at[b], acc_sh.at[dst_v.at[bp, ji]], add=True)
        return ()

    lax.fori_loop(0, CHUNKS, body, ())
    plsc.subcore_barrier()
    for j in range(ROWS_PT // K):
        pltpu.sync_copy(acc_sh.at[pl.ds(row0 + j * K, K)], rows_v.at[j % 2])
        pltpu.sync_copy(rows_v.at[j % 2],
                        out_hbm.at[c, pl.ds(row0 + j * K, K)])


BM = 1024


def _tc_mm_body(x_ref, w_ref, h_ref):
    h_ref[...] = jnp.dot(x_ref[...], w_ref[...],
                         preferred_element_type=jnp.float32)


def _tc_mm(x_p, w1):
    return pl.pallas_call(
        _tc_mm_body,
        grid=(N_PAD // BM,),
        in_specs=[
            pl.BlockSpec((BM, D), lambda i: (i, 0)),
            pl.BlockSpec((D, D), lambda i: (0, 0)),
        ],
        out_specs=pl.BlockSpec((BM, D), lambda i: (i, 0)),
        out_shape=jax.ShapeDtypeStruct((N_PAD, D), jnp.float32),
    )(x_p, w1)


def _tc1_body(h_ref, degp_ref, hp_ref, dinv_ref):
    deg = degp_ref[0, :] + degp_ref[1, :] + 1.0
    dinv = (1.0 / jnp.sqrt(deg))[:, None]
    hp_ref[...] = h_ref[...] * dinv
    dinv_ref[...] = dinv


def _tc1(h, degp):
    return pl.pallas_call(
        _tc1_body,
        grid=(N_PAD // BM,),
        in_specs=[
            pl.BlockSpec((BM, D), lambda i: (i, 0)),
            pl.BlockSpec((NC, BM), lambda i: (0, i)),
        ],
        out_specs=[
            pl.BlockSpec((BM, D), lambda i: (i, 0)),
            pl.BlockSpec((BM, 1), lambda i: (i, 0)),
        ],
        out_shape=[
            jax.ShapeDtypeStruct((N_PAD, D), jnp.float32),
            jax.ShapeDtypeStruct((N_PAD, 1), jnp.float32),
        ],
    )(h, degp)


def _tc2_body(part_ref, hp_ref, dinv_ref, b_ref, w_ref, out_ref):
    t = (part_ref[0] + part_ref[1] + hp_ref[...]) * dinv_ref[...] + b_ref[...]
    t = jnp.maximum(t, 0.0)
    h = jnp.dot(t, w_ref[...], preferred_element_type=jnp.float32)
    out_ref[...] = h * dinv_ref[...]


def _tc2(part, hp, dinv, b, w):
    return pl.pallas_call(
        _tc2_body,
        grid=(N_PAD // BM,),
        in_specs=[
            pl.BlockSpec((NC, BM, D), lambda i: (0, i, 0)),
            pl.BlockSpec((BM, D), lambda i: (i, 0)),
            pl.BlockSpec((BM, 1), lambda i: (i, 0)),
            pl.BlockSpec((1, D), lambda i: (0, 0)),
            pl.BlockSpec((D, D), lambda i: (0, 0)),
        ],
        out_specs=pl.BlockSpec((BM, D), lambda i: (i, 0)),
        out_shape=jax.ShapeDtypeStruct((N_PAD, D), jnp.float32),
    )(part, hp, dinv, b.reshape(1, D), w)


def _tc3_body(part_ref, hp_ref, dinv_ref, b_ref, out_ref):
    out_ref[...] = ((part_ref[0] + part_ref[1] + hp_ref[...]) * dinv_ref[...]
                    + b_ref[...])


def _tc3(part, hp, dinv, b):
    return pl.pallas_call(
        _tc3_body,
        grid=(N_PAD // BM,),
        in_specs=[
            pl.BlockSpec((NC, BM, D), lambda i: (0, i, 0)),
            pl.BlockSpec((BM, D), lambda i: (i, 0)),
            pl.BlockSpec((BM, 1), lambda i: (i, 0)),
            pl.BlockSpec((1, D), lambda i: (0, 0)),
        ],
        out_specs=pl.BlockSpec((BM, D), lambda i: (i, 0)),
        out_shape=jax.ShapeDtypeStruct((N_PAD, D), jnp.float32),
    )(part, hp, dinv, b.reshape(1, D))


def _tc_ques_body(q_ref, w_ref, b_ref, out_ref):
    out_ref[...] = jnp.dot(q_ref[...], w_ref[...],
                           preferred_element_type=jnp.float32) + b_ref[...]


def _tc_ques(q_emb, wq, bq):
    return pl.pallas_call(
        _tc_ques_body,
        out_shape=jax.ShapeDtypeStruct(q_emb.shape, jnp.float32),
    )(q_emb, wq, bq.reshape(1, D))


def kernel(x, edge_index, W1, b1, W2, b2, Wq, bq, q_emb):
    dst = edge_index[1]
    fill = N + jnp.arange(FILL, dtype=jnp.int32)
    dst_pad = jnp.concatenate(
        [dst.reshape(NW, EPT_REAL),
         jnp.broadcast_to(fill, (NW, FILL))], axis=1)
    dst_p = dst_pad.reshape(NW, CHUNKS, K)
    dst_p_wide = dst_pad.reshape(NW, CHUNKS // 2, 2 * K)
    x_p = jnp.pad(x, ((0, N_PAD - N), (0, 0)))

    ones_k = jnp.ones((2 * K,), jnp.float32)
    zeros_k = jnp.zeros((2 * K,), jnp.float32)
    zrows = jnp.zeros((K, D), jnp.float32)

    degp = _sc_degree(dst_p_wide, ones_k, zeros_k)
    h1 = _tc_mm(x_p, W1)
    hp1, dinv = _tc1(h1, degp)
    ei_flat = edge_index.reshape(2 * E)
    part1 = _sc_scatter(hp1, ei_flat, dst_p, fill, zrows)
    hp2 = _tc2(part1, hp1, dinv, b1, W2)
    part2 = _sc_scatter(hp2, ei_flat, dst_p, fill, zrows)
    h2 = _tc3(part2, hp2, dinv, b2)
    ques = _tc_ques(q_emb, Wq, bq)
    return (ques, h2[:N])


# speedup vs baseline: 3.1820x; 3.1820x over previous
import functools

import jax
import jax.numpy as jnp
from jax import lax
from jax.experimental import pallas as pl
from jax.experimental.pallas import tpu as pltpu
from jax.experimental.pallas import tpu_sc as plsc

N = 10000
E = 320000
D = 128

NC = 2
NS = 16
NW = NC * NS

K = 64
CHUNKS = 160
IB = 16
NBUF = 4
EPT = K * CHUNKS
EPT_REAL = E // NW
FILL = EPT - EPT_REAL
N_PAD = 10240
ROWS_PT = N_PAD // NS

_mesh = plsc.VectorSubcoreMesh(
    core_axis_name="c", subcore_axis_name="s", num_cores=NC, num_subcores=NS)


@functools.partial(
    pl.kernel,
    out_type=jax.ShapeDtypeStruct((NC, N_PAD), jnp.float32),
    mesh=_mesh,
    scratch_types=[
        pltpu.VMEM((CHUNKS // 2, 2 * K), jnp.int32),
        pltpu.VMEM((2 * K,), jnp.float32),
        pltpu.VMEM((2 * K,), jnp.float32),
        pltpu.VMEM((ROWS_PT,), jnp.float32),
        pltpu.VMEM_SHARED((N_PAD,), jnp.float32),
    ],
)
def _sc_degree(dst_hbm, ones_hbm, zeros_hbm, deg_out, idx_v, ones_v, zeros_v,
               bounce_v, deg_sh):
    c = lax.axis_index("c")
    s = lax.axis_index("s")
    wid = s * NC + c
    row0 = s * ROWS_PT
    pltpu.sync_copy(ones_hbm, ones_v)
    pltpu.sync_copy(zeros_hbm, zeros_v)
    for j in range(ROWS_PT // (2 * K)):
        pltpu.sync_copy(zeros_v, deg_sh.at[pl.ds(row0 + j * 2 * K, 2 * K)])
    pltpu.sync_copy(dst_hbm.at[wid], idx_v)
    plsc.subcore_barrier()

    def body(j, _):
        pltpu.sync_copy(ones_v, deg_sh.at[idx_v.at[j]], add=True)
        return ()

    lax.fori_loop(0, CHUNKS // 2, body, ())
    plsc.subcore_barrier()
    pltpu.sync_copy(deg_sh.at[pl.ds(row0, ROWS_PT)], bounce_v)
    pltpu.sync_copy(bounce_v, deg_out.at[c, pl.ds(row0, ROWS_PT)])


@functools.partial(
    pl.kernel,
    out_type=jax.ShapeDtypeStruct((NC, N_PAD, D), jnp.float32),
    mesh=_mesh,
    scratch_types=[
        pltpu.VMEM((EPT,), jnp.int32),
        pltpu.VMEM((2, IB, K), jnp.int32),
        pltpu.VMEM((NBUF, K, D), jnp.float32),
        pltpu.VMEM_SHARED((N_PAD, D), jnp.float32),
        pltpu.SemaphoreType.DMA((NBUF,)),
        pltpu.SemaphoreType.DMA,
    ],
)
def _sc_scatter(hp_hbm, ei_hbm, dst_hbm, fill_hbm, zrows_hbm, out_hbm, src_v,
                dst_v, rows_v, acc_sh, sems, sem_d):
    c = lax.axis_index("c")
    s = lax.axis_index("s")
    wid = s * NC + c
    row0 = s * ROWS_PT

    def dst_block_copy(jb, bp):
        return pltpu.make_async_copy(
            dst_hbm.at[wid, pl.ds(jb * IB, IB)], dst_v.at[bp], sem_d)

    def gather(j, b):
        idx = src_v.at[pl.ds(j * K, K)]
        return pltpu.make_async_copy(
            hp_hbm.at[idx], rows_v.at[b], sems.at[b])

    pltpu.sync_copy(zrows_hbm, rows_v.at[0])
    zstores = [
        pltpu.make_async_copy(rows_v.at[0],
                              acc_sh.at[pl.ds(row0 + j * K, K)], sems.at[0])
        for j in range(ROWS_PT // K)
    ]
    for z in zstores:
        z.start()
    pltpu.sync_copy(ei_hbm.at[pl.ds(wid * EPT_REAL, EPT_REAL)],
                    src_v.at[pl.ds(0, EPT_REAL)])
    pltpu.sync_copy(fill_hbm, src_v.at[pl.ds(EPT_REAL, EPT - EPT_REAL)])
    dst_block_copy(0, 0).start()
    for z in zstores:
        z.wait()
    plsc.subcore_barrier()

    for b in range(NBUF - 1):
        gather(b, b).start()
    dst_block_copy(0, 0).wait()
    dst_block_copy(1, 1).start()

    def body(j, _):
        b = lax.rem(j, NBUF)
        jb = lax.div(j, IB)
        ji = lax.rem(j, IB)
        bp = lax.rem(jb, 2)

        @pl.when(j + NBUF - 1 < CHUNKS)
        def _():
            gather(j + NBUF - 1, lax.rem(j + NBUF - 1, NBUF)).start()

        gather(j, b).wait()

        @pl.when((ji == 0) & (jb >= 1))
        def _():
            dst_block_copy(jb, bp).wait()

            @pl.when(jb + 1 < CHUNKS // IB)
            def _():
                dst_block_copy(jb + 1, 1 - bp).start()

        pltpu.sync_copy(rows_v.at[b], acc_sh.at[dst_v.at[bp, ji]], add=True)
        return ()

    lax.fori_loop(0, CHUNKS, body, ())
    plsc.subcore_barrier()
    for j in range(ROWS_PT // K):
        pltpu.sync_copy(acc_sh.at[pl.ds(row0 + j * K, K)], rows_v.at[j % 2])
        pltpu.sync_copy(rows_v.at[j % 2],
                        out_hbm.at[c, pl.ds(row0 + j * K, K)])


BM = 1024


def _tc_mm_body(x_ref, w_ref, h_ref):
    h_ref[...] = jnp.dot(x_ref[...], w_ref[...],
                         preferred_element_type=jnp.float32)


def _tc_mm(x_p, w1):
    return pl.pallas_call(
        _tc_mm_body,
        grid=(N_PAD // BM,),
        in_specs=[
            pl.BlockSpec((BM, D), lambda i: (i, 0)),
            pl.BlockSpec((D, D), lambda i: (0, 0)),
        ],
        out_specs=pl.BlockSpec((BM, D), lambda i: (i, 0)),
        out_shape=jax.ShapeDtypeStruct((N_PAD, D), jnp.float32),
    )(x_p, w1)


def _tc1_body(h_ref, degp_ref, hp_ref, dinv_ref):
    deg = degp_ref[0, :] + degp_ref[1, :] + 1.0
    dinv = (1.0 / jnp.sqrt(deg))[:, None]
    hp_ref[...] = h_ref[...] * dinv
    dinv_ref[...] = dinv


def _tc1(h, degp):
    return pl.pallas_call(
        _tc1_body,
        grid=(N_PAD // BM,),
        in_specs=[
            pl.BlockSpec((BM, D), lambda i: (i, 0)),
            pl.BlockSpec((NC, BM), lambda i: (0, i)),
        ],
        out_specs=[
            pl.BlockSpec((BM, D), lambda i: (i, 0)),
            pl.BlockSpec((BM, 1), lambda i: (i, 0)),
        ],
        out_shape=[
            jax.ShapeDtypeStruct((N_PAD, D), jnp.float32),
            jax.ShapeDtypeStruct((N_PAD, 1), jnp.float32),
        ],
    )(h, degp)


def _tc2_body(part_ref, hp_ref, dinv_ref, b_ref, w_ref, out_ref):
    t = (part_ref[0] + part_ref[1] + hp_ref[...]) * dinv_ref[...] + b_ref[...]
    t = jnp.maximum(t, 0.0)
    h = jnp.dot(t, w_ref[...], preferred_element_type=jnp.float32)
    out_ref[...] = h * dinv_ref[...]


def _tc2(part, hp, dinv, b, w):
    return pl.pallas_call(
        _tc2_body,
        grid=(N_PAD // BM,),
        in_specs=[
            pl.BlockSpec((NC, BM, D), lambda i: (0, i, 0)),
            pl.BlockSpec((BM, D), lambda i: (i, 0)),
            pl.BlockSpec((BM, 1), lambda i: (i, 0)),
            pl.BlockSpec((1, D), lambda i: (0, 0)),
            pl.BlockSpec((D, D), lambda i: (0, 0)),
        ],
        out_specs=pl.BlockSpec((BM, D), lambda i: (i, 0)),
        out_shape=jax.ShapeDtypeStruct((N_PAD, D), jnp.float32),
    )(part, hp, dinv, b.reshape(1, D), w)


def _tc3_body(part_ref, hp_ref, dinv_ref, b_ref, out_ref):
    out_ref[...] = ((part_ref[0] + part_ref[1] + hp_ref[...]) * dinv_ref[...]
                    + b_ref[...])


def _tc3(part, hp, dinv, b):
    return pl.pallas_call(
        _tc3_body,
        grid=(N_PAD // BM,),
        in_specs=[
            pl.BlockSpec((NC, BM, D), lambda i: (0, i, 0)),
            pl.BlockSpec((BM, D), lambda i: (i, 0)),
            pl.BlockSpec((BM, 1), lambda i: (i, 0)),
            pl.BlockSpec((1, D), lambda i: (0, 0)),
        ],
        out_specs=pl.BlockSpec((BM, D), lambda i: (i, 0)),
        out_shape=jax.ShapeDtypeStruct((N_PAD, D), jnp.float32),
    )(part, hp, dinv, b.reshape(1, D))


def _tc_ques_body(q_ref, w_ref, b_ref, out_ref):
    out_ref[...] = jnp.dot(q_ref[...], w_ref[...],
                           preferred_element_type=jnp.float32) + b_ref[...]


def _tc_ques(q_emb, wq, bq):
    return pl.pallas_call(
        _tc_ques_body,
        out_shape=jax.ShapeDtypeStruct(q_emb.shape, jnp.float32),
    )(q_emb, wq, bq.reshape(1, D))


def kernel(x, edge_index, W1, b1, W2, b2, Wq, bq, q_emb):
    dst = edge_index[1]
    fill = N + jnp.arange(FILL, dtype=jnp.int32)
    dst_pad = jnp.concatenate(
        [dst.reshape(NW, EPT_REAL),
         jnp.broadcast_to(fill, (NW, FILL))], axis=1)
    dst_p = dst_pad.reshape(NW, CHUNKS, K)
    dst_p_wide = dst_pad.reshape(NW, CHUNKS // 2, 2 * K)
    x_p = jnp.pad(x, ((0, N_PAD - N), (0, 0)))

    ones_k = jnp.ones((2 * K,), jnp.float32)
    zeros_k = jnp.zeros((2 * K,), jnp.float32)
    zrows = jnp.zeros((K, D), jnp.float32)

    degp = _sc_degree(dst_p_wide, ones_k, zeros_k)
    h1 = _tc_mm(x_p, W1)
    hp1, dinv = _tc1(h1, degp)
    src_flat = edge_index[0]
    part1 = _sc_scatter(hp1, src_flat, dst_p, fill, zrows)
    hp2 = _tc2(part1, hp1, dinv, b1, W2)
    part2 = _sc_scatter(hp2, src_flat, dst_p, fill, zrows)
    h2 = _tc3(part2, hp2, dinv, b2)
    ques = _tc_ques(q_emb, Wq, bq)
    return (ques, h2[:N])


# speedup vs baseline: 3.2449x; 1.0198x over previous
import functools

import jax
import jax.numpy as jnp
from jax import lax
from jax.experimental import pallas as pl
from jax.experimental.pallas import tpu as pltpu
from jax.experimental.pallas import tpu_sc as plsc

N = 10000
E = 320000
D = 128

NC = 2
NS = 16
NW = NC * NS

K = 64
CHUNKS = 160
IB = 16
NBUF = 4
EPT = K * CHUNKS
EPT_REAL = E // NW
FILL = EPT - EPT_REAL
N_PAD = 10240
ROWS_PT = N_PAD // NS

_mesh = plsc.VectorSubcoreMesh(
    core_axis_name="c", subcore_axis_name="s", num_cores=NC, num_subcores=NS)


@functools.partial(
    pl.kernel,
    out_type=jax.ShapeDtypeStruct((NC, N_PAD), jnp.float32),
    mesh=_mesh,
    scratch_types=[
        pltpu.VMEM((CHUNKS // 2, 2 * K), jnp.int32),
        pltpu.VMEM((2 * K,), jnp.float32),
        pltpu.VMEM((2 * K,), jnp.float32),
        pltpu.VMEM((ROWS_PT,), jnp.float32),
        pltpu.VMEM_SHARED((N_PAD,), jnp.float32),
    ],
)
def _sc_degree(dst_hbm, ones_hbm, zeros_hbm, deg_out, idx_v, ones_v, zeros_v,
               bounce_v, deg_sh):
    c = lax.axis_index("c")
    s = lax.axis_index("s")
    wid = s * NC + c
    row0 = s * ROWS_PT
    pltpu.sync_copy(ones_hbm, ones_v)
    pltpu.sync_copy(zeros_hbm, zeros_v)
    for j in range(ROWS_PT // (2 * K)):
        pltpu.sync_copy(zeros_v, deg_sh.at[pl.ds(row0 + j * 2 * K, 2 * K)])
    pltpu.sync_copy(dst_hbm.at[wid], idx_v)
    plsc.subcore_barrier()

    def body(j, _):
        pltpu.sync_copy(ones_v, deg_sh.at[idx_v.at[j]], add=True)
        return ()

    lax.fori_loop(0, CHUNKS // 2, body, ())
    plsc.subcore_barrier()
    pltpu.sync_copy(deg_sh.at[pl.ds(row0, ROWS_PT)], bounce_v)
    pltpu.sync_copy(bounce_v, deg_out.at[c, pl.ds(row0, ROWS_PT)])


@functools.partial(
    pl.kernel,
    out_type=jax.ShapeDtypeStruct((NC, N_PAD, D), jnp.float32),
    mesh=_mesh,
    scratch_types=[
        pltpu.VMEM((EPT,), jnp.int32),
        pltpu.VMEM((2, IB, K), jnp.int32),
        pltpu.VMEM((NBUF, K, D), jnp.float32),
        pltpu.VMEM_SHARED((N_PAD, D), jnp.float32),
        pltpu.SemaphoreType.DMA((NBUF,)),
        pltpu.SemaphoreType.DMA,
    ],
)
def _sc_scatter(hp_hbm, ei_hbm, dst_hbm, fill_hbm, zrows_hbm, out_hbm, src_v,
                dst_v, rows_v, acc_sh, sems, sem_d):
    c = lax.axis_index("c")
    s = lax.axis_index("s")
    wid = s * NC + c
    row0 = s * ROWS_PT

    def dst_block_copy(jb, bp):
        return pltpu.make_async_copy(
            dst_hbm.at[wid, pl.ds(jb * IB, IB)], dst_v.at[bp], sem_d)

    def gather(j, b):
        idx = src_v.at[pl.ds(j * K, K)]
        return pltpu.make_async_copy(
            hp_hbm.at[idx], rows_v.at[b], sems.at[b])

    pltpu.sync_copy(zrows_hbm, rows_v.at[0])
    zstores = [
        pltpu.make_async_copy(rows_v.at[0],
                              acc_sh.at[pl.ds(row0 + j * K, K)], sems.at[0])
        for j in range(ROWS_PT // K)
    ]
    for z in zstores:
        z.start()
    pltpu.sync_copy(ei_hbm.at[pl.ds(wid * EPT_REAL, EPT_REAL)],
                    src_v.at[pl.ds(0, EPT_REAL)])
    pltpu.sync_copy(fill_hbm, src_v.at[pl.ds(EPT_REAL, EPT - EPT_REAL)])
    dst_block_copy(0, 0).start()
    for z in zstores:
        z.wait()
    plsc.subcore_barrier()

    for b in range(NBUF - 1):
        gather(b, b).start()
    dst_block_copy(0, 0).wait()
    dst_block_copy(1, 1).start()

    def body(j, _):
        b = lax.rem(j, NBUF)
        jb = lax.div(j, IB)
        ji = lax.rem(j, IB)
        bp = lax.rem(jb, 2)

        @pl.when(j + NBUF - 1 < CHUNKS)
        def _():
            gather(j + NBUF - 1, lax.rem(j + NBUF - 1, NBUF)).start()

        gather(j, b).wait()

        @pl.when((ji == 0) & (jb >= 1))
        def _():
            dst_block_copy(jb, bp).wait()

            @pl.when(jb + 1 < CHUNKS // IB)
            def _():
                dst_block_copy(jb + 1, 1 - bp).start()

        pltpu.sync_copy(rows_v.at[b], acc_sh.at[dst_v.at[bp, ji]], add=True)
        return ()

    lax.fori_loop(0, CHUNKS, body, ())
    plsc.subcore_barrier()

    def w_st(j, sl):
        return pltpu.make_async_copy(
            acc_sh.at[pl.ds(row0 + j * K, K)], rows_v.at[sl], sems.at[sl])

    def w_th(j, sl):
        return pltpu.make_async_copy(
            rows_v.at[sl], out_hbm.at[c, pl.ds(row0 + j * K, K)],
            sems.at[2 + sl])

    n_wo = ROWS_PT // K
    w_st(0, 0).start()
    for j in range(n_wo):
        sl = j % 2
        w_st(j, sl).wait()
        if j + 1 < n_wo:
            if j >= 1:
                w_th(j - 1, (j - 1) % 2).wait()
            w_st(j + 1, (j + 1) % 2).start()
        w_th(j, sl).start()
    w_th(n_wo - 2, (n_wo - 2) % 2).wait()
    w_th(n_wo - 1, (n_wo - 1) % 2).wait()


BM = 1024


def _tc_mm_body(x_ref, w_ref, h_ref):
    h_ref[...] = jnp.dot(x_ref[...], w_ref[...],
                         preferred_element_type=jnp.float32)


def _tc_mm(x_p, w1):
    return pl.pallas_call(
        _tc_mm_body,
        grid=(N_PAD // BM,),
        in_specs=[
            pl.BlockSpec((BM, D), lambda i: (i, 0)),
            pl.BlockSpec((D, D), lambda i: (0, 0)),
        ],
        out_specs=pl.BlockSpec((BM, D), lambda i: (i, 0)),
        out_shape=jax.ShapeDtypeStruct((N_PAD, D), jnp.float32),
    )(x_p, w1)


def _tc1_body(h_ref, degp_ref, hp_ref, dinv_ref):
    deg = degp_ref[0, :] + degp_ref[1, :] + 1.0
    dinv = (1.0 / jnp.sqrt(deg))[:, None]
    hp_ref[...] = h_ref[...] * dinv
    dinv_ref[...] = dinv


def _tc1(h, degp):
    return pl.pallas_call(
        _tc1_body,
        grid=(N_PAD // BM,),
        in_specs=[
            pl.BlockSpec((BM, D), lambda i: (i, 0)),
            pl.BlockSpec((NC, BM), lambda i: (0, i)),
        ],
        out_specs=[
            pl.BlockSpec((BM, D), lambda i: (i, 0)),
            pl.BlockSpec((BM, 1), lambda i: (i, 0)),
        ],
        out_shape=[
            jax.ShapeDtypeStruct((N_PAD, D), jnp.float32),
            jax.ShapeDtypeStruct((N_PAD, 1), jnp.float32),
        ],
    )(h, degp)


def _tc2_body(part_ref, hp_ref, dinv_ref, b_ref, w_ref, out_ref):
    t = (part_ref[0] + part_ref[1] + hp_ref[...]) * dinv_ref[...] + b_ref[...]
    t = jnp.maximum(t, 0.0)
    h = jnp.dot(t, w_ref[...], preferred_element_type=jnp.float32)
    out_ref[...] = h * dinv_ref[...]


def _tc2(part, hp, dinv, b, w):
    return pl.pallas_call(
        _tc2_body,
        grid=(N_PAD // BM,),
        in_specs=[
            pl.BlockSpec((NC, BM, D), lambda i: (0, i, 0)),
            pl.BlockSpec((BM, D), lambda i: (i, 0)),
            pl.BlockSpec((BM, 1), lambda i: (i, 0)),
            pl.BlockSpec((1, D), lambda i: (0, 0)),
            pl.BlockSpec((D, D), lambda i: (0, 0)),
        ],
        out_specs=pl.BlockSpec((BM, D), lambda i: (i, 0)),
        out_shape=jax.ShapeDtypeStruct((N_PAD, D), jnp.float32),
    )(part, hp, dinv, b.reshape(1, D), w)


def _tc3_body(part_ref, hp_ref, dinv_ref, b_ref, out_ref):
    out_ref[...] = ((part_ref[0] + part_ref[1] + hp_ref[...]) * dinv_ref[...]
                    + b_ref[...])


def _tc3(part, hp, dinv, b):
    return pl.pallas_call(
        _tc3_body,
        grid=(N_PAD // BM,),
        in_specs=[
            pl.BlockSpec((NC, BM, D), lambda i: (0, i, 0)),
            pl.BlockSpec((BM, D), lambda i: (i, 0)),
            pl.BlockSpec((BM, 1), lambda i: (i, 0)),
            pl.BlockSpec((1, D), lambda i: (0, 0)),
        ],
        out_specs=pl.BlockSpec((BM, D), lambda i: (i, 0)),
        out_shape=jax.ShapeDtypeStruct((N_PAD, D), jnp.float32),
    )(part, hp, dinv, b.reshape(1, D))


def _tc_ques_body(q_ref, w_ref, b_ref, out_ref):
    out_ref[...] = jnp.dot(q_ref[...], w_ref[...],
                           preferred_element_type=jnp.float32) + b_ref[...]


def _tc_ques(q_emb, wq, bq):
    return pl.pallas_call(
        _tc_ques_body,
        out_shape=jax.ShapeDtypeStruct(q_emb.shape, jnp.float32),
    )(q_emb, wq, bq.reshape(1, D))


def kernel(x, edge_index, W1, b1, W2, b2, Wq, bq, q_emb):
    dst = edge_index[1]
    fill = N + jnp.arange(FILL, dtype=jnp.int32)
    dst_pad = jnp.concatenate(
        [dst.reshape(NW, EPT_REAL),
         jnp.broadcast_to(fill, (NW, FILL))], axis=1)
    dst_p = dst_pad.reshape(NW, CHUNKS, K)
    dst_p_wide = dst_pad.reshape(NW, CHUNKS // 2, 2 * K)
    x_p = jnp.pad(x, ((0, N_PAD - N), (0, 0)))

    ones_k = jnp.ones((2 * K,), jnp.float32)
    zeros_k = jnp.zeros((2 * K,), jnp.float32)
    zrows = jnp.zeros((K, D), jnp.float32)

    degp = _sc_degree(dst_p_wide, ones_k, zeros_k)
    h1 = _tc_mm(x_p, W1)
    hp1, dinv = _tc1(h1, degp)
    src_flat = edge_index[0]
    part1 = _sc_scatter(hp1, src_flat, dst_p, fill, zrows)
    hp2 = _tc2(part1, hp1, dinv, b1, W2)
    part2 = _sc_scatter(hp2, src_flat, dst_p, fill, zrows)
    h2 = _tc3(part2, hp2, dinv, b2)
    ques = _tc_ques(q_emb, Wq, bq)
    return (ques, h2[:N])


# speedup vs baseline: 3.2956x; 1.0156x over previous
import functools

import jax
import jax.numpy as jnp
from jax import lax
from jax.experimental import pallas as pl
from jax.experimental.pallas import tpu as pltpu
from jax.experimental.pallas import tpu_sc as plsc

N = 10000
E = 320000
D = 128

NC = 2
NS = 16
NW = NC * NS

K = 64
CHUNKS = 160
IB = 16
NBUF = 4
EPT = K * CHUNKS
EPT_REAL = E // NW
FILL = EPT - EPT_REAL
N_PAD = 10240
ROWS_PT = N_PAD // NS

_mesh = plsc.VectorSubcoreMesh(
    core_axis_name="c", subcore_axis_name="s", num_cores=NC, num_subcores=NS)


@functools.partial(
    pl.kernel,
    out_type=jax.ShapeDtypeStruct((NC, N_PAD), jnp.float32),
    mesh=_mesh,
    scratch_types=[
        pltpu.VMEM((CHUNKS // 2, 2 * K), jnp.int32),
        pltpu.VMEM((2 * K,), jnp.float32),
        pltpu.VMEM((2 * K,), jnp.float32),
        pltpu.VMEM((ROWS_PT,), jnp.float32),
        pltpu.VMEM_SHARED((N_PAD,), jnp.float32),
    ],
)
def _sc_degree(dst_hbm, ones_hbm, zeros_hbm, deg_out, idx_v, ones_v, zeros_v,
               bounce_v, deg_sh):
    c = lax.axis_index("c")
    s = lax.axis_index("s")
    wid = s * NC + c
    row0 = s * ROWS_PT
    pltpu.sync_copy(ones_hbm, ones_v)
    pltpu.sync_copy(zeros_hbm, zeros_v)
    for j in range(ROWS_PT // (2 * K)):
        pltpu.sync_copy(zeros_v, deg_sh.at[pl.ds(row0 + j * 2 * K, 2 * K)])
    pltpu.sync_copy(dst_hbm.at[wid], idx_v)
    plsc.subcore_barrier()

    def body(j, _):
        pltpu.sync_copy(ones_v, deg_sh.at[idx_v.at[j]], add=True)
        return ()

    lax.fori_loop(0, CHUNKS // 2, body, ())
    plsc.subcore_barrier()
    pltpu.sync_copy(deg_sh.at[pl.ds(row0, ROWS_PT)], bounce_v)
    pltpu.sync_copy(bounce_v, deg_out.at[c, pl.ds(row0, ROWS_PT)])


@functools.partial(
    pl.kernel,
    out_type=jax.ShapeDtypeStruct((NC, N_PAD, D), jnp.float32),
    mesh=_mesh,
    scratch_types=[
        pltpu.VMEM((EPT,), jnp.int32),
        pltpu.VMEM((2, IB, K), jnp.int32),
        pltpu.VMEM((NBUF, K, D), jnp.float32),
        pltpu.VMEM_SHARED((N_PAD, D), jnp.float32),
        pltpu.SemaphoreType.DMA((NBUF,)),
        pltpu.SemaphoreType.DMA,
    ],
)
def _sc_scatter(hp_hbm, ei_hbm, dst_hbm, fill_hbm, zrows_hbm, out_hbm, src_v,
                dst_v, rows_v, acc_sh, sems, sem_d):
    c = lax.axis_index("c")
    s = lax.axis_index("s")
    wid = s * NC + c
    row0 = s * ROWS_PT

    def dst_block_copy(jb, bp):
        return pltpu.make_async_copy(
            dst_hbm.at[wid, pl.ds(jb * IB, IB)], dst_v.at[bp], sem_d)

    def gather(j, b):
        idx = src_v.at[pl.ds(j * K, K)]
        return pltpu.make_async_copy(
            hp_hbm.at[idx], rows_v.at[b], sems.at[b])

    pltpu.sync_copy(zrows_hbm, rows_v.at[0])
    zstores = [
        pltpu.make_async_copy(rows_v.at[0],
                              acc_sh.at[pl.ds(row0 + j * K, K)], sems.at[0])
        for j in range(ROWS_PT // K)
    ]
    for z in zstores:
        z.start()
    pltpu.sync_copy(ei_hbm.at[pl.ds(wid * EPT_REAL, EPT_REAL)],
                    src_v.at[pl.ds(0, EPT_REAL)])
    pltpu.sync_copy(fill_hbm, src_v.at[pl.ds(EPT_REAL, EPT - EPT_REAL)])
    dst_block_copy(0, 0).start()
    for z in zstores:
        z.wait()
    plsc.subcore_barrier()

    for b in range(NBUF - 1):
        gather(b, b).start()
    dst_block_copy(0, 0).wait()
    dst_block_copy(1, 1).start()

    def body(j, _):
        b = lax.rem(j, NBUF)
        jb = lax.div(j, IB)
        ji = lax.rem(j, IB)
        bp = lax.rem(jb, 2)

        @pl.when(j + NBUF - 1 < CHUNKS)
        def _():
            gather(j + NBUF - 1, lax.rem(j + NBUF - 1, NBUF)).start()

        gather(j, b).wait()

        @pl.when((ji == 0) & (jb >= 1))
        def _():
            dst_block_copy(jb, bp).wait()

            @pl.when(jb + 1 < CHUNKS // IB)
            def _():
                dst_block_copy(jb + 1, 1 - bp).start()

        pltpu.sync_copy(rows_v.at[b], acc_sh.at[dst_v.at[bp, ji]], add=True)
        return ()

    lax.fori_loop(0, CHUNKS, body, ())
    plsc.subcore_barrier()

    def w_st(j, sl):
        return pltpu.make_async_copy(
            acc_sh.at[pl.ds(row0 + j * K, K)], rows_v.at[sl], sems.at[sl])

    def w_th(j, sl):
        return pltpu.make_async_copy(
            rows_v.at[sl], out_hbm.at[c, pl.ds(row0 + j * K, K)],
            sems.at[2 + sl])

    n_wo = ROWS_PT // K
    w_st(0, 0).start()
    for j in range(n_wo):
        sl = j % 2
        w_st(j, sl).wait()
        if j + 1 < n_wo:
            if j >= 1:
                w_th(j - 1, (j - 1) % 2).wait()
            w_st(j + 1, (j + 1) % 2).start()
        w_th(j, sl).start()
    w_th(n_wo - 2, (n_wo - 2) % 2).wait()
    w_th(n_wo - 1, (n_wo - 1) % 2).wait()


BM = 1024


def _tc_mm_body(x_ref, w_ref, h_ref):
    h_ref[...] = jnp.dot(x_ref[...], w_ref[...],
                         preferred_element_type=jnp.float32)


def _tc_mm(x_p, w1):
    return pl.pallas_call(
        _tc_mm_body,
        grid=(N_PAD // BM,),
        in_specs=[
            pl.BlockSpec((BM, D), lambda i: (i, 0)),
            pl.BlockSpec((D, D), lambda i: (0, 0)),
        ],
        out_specs=pl.BlockSpec((BM, D), lambda i: (i, 0)),
        out_shape=jax.ShapeDtypeStruct((N_PAD, D), jnp.float32),
    )(x_p, w1)


def _tc1_body(h_ref, degp_ref, hp_ref, dinv_ref):
    deg = degp_ref[0, :] + degp_ref[1, :] + 1.0
    dinv = (1.0 / jnp.sqrt(deg))[:, None]
    hp_ref[...] = h_ref[...] * dinv
    dinv_ref[...] = dinv


def _tc1(h, degp):
    return pl.pallas_call(
        _tc1_body,
        grid=(N_PAD // BM,),
        in_specs=[
            pl.BlockSpec((BM, D), lambda i: (i, 0)),
            pl.BlockSpec((NC, BM), lambda i: (0, i)),
        ],
        out_specs=[
            pl.BlockSpec((BM, D), lambda i: (i, 0)),
            pl.BlockSpec((BM, 1), lambda i: (i, 0)),
        ],
        out_shape=[
            jax.ShapeDtypeStruct((N_PAD, D), jnp.float32),
            jax.ShapeDtypeStruct((N_PAD, 1), jnp.float32),
        ],
    )(h, degp)


def _tc2_body(part_ref, hp_ref, dinv_ref, b_ref, w_ref, out_ref):
    t = (part_ref[0] + part_ref[1] + hp_ref[...]) * dinv_ref[...] + b_ref[...]
    t = jnp.maximum(t, 0.0)
    h = jnp.dot(t, w_ref[...], preferred_element_type=jnp.float32)
    out_ref[...] = h * dinv_ref[...]


def _tc2(part, hp, dinv, b, w):
    return pl.pallas_call(
        _tc2_body,
        grid=(N_PAD // BM,),
        in_specs=[
            pl.BlockSpec((NC, BM, D), lambda i: (0, i, 0)),
            pl.BlockSpec((BM, D), lambda i: (i, 0)),
            pl.BlockSpec((BM, 1), lambda i: (i, 0)),
            pl.BlockSpec((1, D), lambda i: (0, 0)),
            pl.BlockSpec((D, D), lambda i: (0, 0)),
        ],
        out_specs=pl.BlockSpec((BM, D), lambda i: (i, 0)),
        out_shape=jax.ShapeDtypeStruct((N_PAD, D), jnp.float32),
    )(part, hp, dinv, b.reshape(1, D), w)


def _tc3_body(part_ref, hp_ref, dinv_ref, b_ref, out_ref):
    out_ref[...] = ((part_ref[0] + part_ref[1] + hp_ref[...]) * dinv_ref[...]
                    + b_ref[...])


def _tc3(part, hp, dinv, b):
    bm = 1000
    return pl.pallas_call(
        _tc3_body,
        grid=(N // bm,),
        in_specs=[
            pl.BlockSpec((NC, bm, D), lambda i: (0, i, 0)),
            pl.BlockSpec((bm, D), lambda i: (i, 0)),
            pl.BlockSpec((bm, 1), lambda i: (i, 0)),
            pl.BlockSpec((1, D), lambda i: (0, 0)),
        ],
        out_specs=pl.BlockSpec((bm, D), lambda i: (i, 0)),
        out_shape=jax.ShapeDtypeStruct((N, D), jnp.float32),
    )(part, hp, dinv, b.reshape(1, D))


def _tc_ques_body(q_ref, w_ref, b_ref, out_ref):
    out_ref[...] = jnp.dot(q_ref[...], w_ref[...],
                           preferred_element_type=jnp.float32) + b_ref[...]


def _tc_ques(q_emb, wq, bq):
    return pl.pallas_call(
        _tc_ques_body,
        out_shape=jax.ShapeDtypeStruct(q_emb.shape, jnp.float32),
    )(q_emb, wq, bq.reshape(1, D))


def kernel(x, edge_index, W1, b1, W2, b2, Wq, bq, q_emb):
    dst = edge_index[1]
    fill = N + jnp.arange(FILL, dtype=jnp.int32)
    dst_pad = jnp.concatenate(
        [dst.reshape(NW, EPT_REAL),
         jnp.broadcast_to(fill, (NW, FILL))], axis=1)
    dst_p = dst_pad.reshape(NW, CHUNKS, K)
    dst_p_wide = dst_pad.reshape(NW, CHUNKS // 2, 2 * K)
    x_p = jnp.pad(x, ((0, N_PAD - N), (0, 0)))

    ones_k = jnp.ones((2 * K,), jnp.float32)
    zeros_k = jnp.zeros((2 * K,), jnp.float32)
    zrows = jnp.zeros((K, D), jnp.float32)

    degp = _sc_degree(dst_p_wide, ones_k, zeros_k)
    h1 = _tc_mm(x_p, W1)
    hp1, dinv = _tc1(h1, degp)
    src_flat = edge_index[0]
    part1 = _sc_scatter(hp1, src_flat, dst_p, fill, zrows)
    hp2 = _tc2(part1, hp1, dinv, b1, W2)
    part2 = _sc_scatter(hp2, src_flat, dst_p, fill, zrows)
    h2 = _tc3(part2, hp2, dinv, b2)
    ques = _tc_ques(q_emb, Wq, bq)
    return (ques, h2)


# speedup vs baseline: 3.3503x; 1.0166x over previous
import functools

import jax
import jax.numpy as jnp
from jax import lax
from jax.experimental import pallas as pl
from jax.experimental.pallas import tpu as pltpu
from jax.experimental.pallas import tpu_sc as plsc

N = 10000
E = 320000
D = 128

NC = 2
NS = 16
NW = NC * NS

K = 64
CHUNKS = 160
IB = 16
NBUF = 4
EPT = K * CHUNKS
EPT_REAL = E // NW
FILL = EPT - EPT_REAL
N_PAD = 10240
ROWS_PT = N_PAD // NS

_mesh = plsc.VectorSubcoreMesh(
    core_axis_name="c", subcore_axis_name="s", num_cores=NC, num_subcores=NS)


@functools.partial(
    pl.kernel,
    out_type=jax.ShapeDtypeStruct((NC, N_PAD), jnp.float32),
    mesh=_mesh,
    scratch_types=[
        pltpu.VMEM((CHUNKS // 2, 2 * K), jnp.int32),
        pltpu.VMEM((2 * K,), jnp.float32),
        pltpu.VMEM((2 * K,), jnp.float32),
        pltpu.VMEM((ROWS_PT,), jnp.float32),
        pltpu.VMEM_SHARED((N_PAD,), jnp.float32),
    ],
)
def _sc_degree(dst_hbm, ones_hbm, zeros_hbm, deg_out, idx_v, ones_v, zeros_v,
               bounce_v, deg_sh):
    c = lax.axis_index("c")
    s = lax.axis_index("s")
    wid = s * NC + c
    row0 = s * ROWS_PT
    pltpu.sync_copy(ones_hbm, ones_v)
    pltpu.sync_copy(zeros_hbm, zeros_v)
    for j in range(ROWS_PT // (2 * K)):
        pltpu.sync_copy(zeros_v, deg_sh.at[pl.ds(row0 + j * 2 * K, 2 * K)])
    pltpu.sync_copy(dst_hbm.at[wid], idx_v)
    plsc.subcore_barrier()

    def body(j, _):
        pltpu.sync_copy(ones_v, deg_sh.at[idx_v.at[j]], add=True)
        return ()

    lax.fori_loop(0, CHUNKS // 2, body, ())
    plsc.subcore_barrier()
    pltpu.sync_copy(deg_sh.at[pl.ds(row0, ROWS_PT)], bounce_v)
    pltpu.sync_copy(bounce_v, deg_out.at[c, pl.ds(row0, ROWS_PT)])


@functools.partial(
    pl.kernel,
    out_type=jax.ShapeDtypeStruct((NC, N_PAD, D), jnp.float32),
    mesh=_mesh,
    scratch_types=[
        pltpu.VMEM((EPT,), jnp.int32),
        pltpu.VMEM((2, IB, K), jnp.int32),
        pltpu.VMEM((NBUF, K, D), jnp.float32),
        pltpu.VMEM_SHARED((N_PAD, D), jnp.float32),
        pltpu.SemaphoreType.DMA((NBUF,)),
        pltpu.SemaphoreType.DMA,
    ],
)
def _sc_scatter(hp_hbm, ei_hbm, dst_hbm, fill_hbm, zrows_hbm, out_hbm, src_v,
                dst_v, rows_v, acc_sh, sems, sem_d):
    c = lax.axis_index("c")
    s = lax.axis_index("s")
    wid = s * NC + c
    row0 = s * ROWS_PT

    def dst_block_copy(jb, bp):
        return pltpu.make_async_copy(
            dst_hbm.at[wid, pl.ds(jb * IB, IB)], dst_v.at[bp], sem_d)

    def gather(j, b):
        idx = src_v.at[pl.ds(j * K, K)]
        return pltpu.make_async_copy(
            hp_hbm.at[idx], rows_v.at[b], sems.at[b])

    pltpu.sync_copy(zrows_hbm, rows_v.at[0])
    zstores = [
        pltpu.make_async_copy(rows_v.at[0],
                              acc_sh.at[pl.ds(row0 + j * K, K)], sems.at[0])
        for j in range(ROWS_PT // K)
    ]
    for z in zstores:
        z.start()
    pltpu.sync_copy(ei_hbm.at[pl.ds(wid * EPT_REAL, EPT_REAL)],
                    src_v.at[pl.ds(0, EPT_REAL)])
    pltpu.sync_copy(fill_hbm, src_v.at[pl.ds(EPT_REAL, EPT - EPT_REAL)])
    dst_block_copy(0, 0).start()
    for z in zstores:
        z.wait()
    plsc.subcore_barrier()

    for b in range(NBUF - 1):
        gather(b, b).start()
    dst_block_copy(0, 0).wait()
    dst_block_copy(1, 1).start()

    def body(j, _):
        b = lax.rem(j, NBUF)
        jb = lax.div(j, IB)
        ji = lax.rem(j, IB)
        bp = lax.rem(jb, 2)

        @pl.when(j + NBUF - 1 < CHUNKS)
        def _():
            gather(j + NBUF - 1, lax.rem(j + NBUF - 1, NBUF)).start()

        gather(j, b).wait()

        @pl.when((ji == 0) & (jb >= 1))
        def _():
            dst_block_copy(jb, bp).wait()

            @pl.when(jb + 1 < CHUNKS // IB)
            def _():
                dst_block_copy(jb + 1, 1 - bp).start()

        pltpu.sync_copy(rows_v.at[b], acc_sh.at[dst_v.at[bp, ji]], add=True)
        return ()

    lax.fori_loop(0, CHUNKS, body, ())
    plsc.subcore_barrier()

    def w_st(j, sl):
        return pltpu.make_async_copy(
            acc_sh.at[pl.ds(row0 + j * K, K)], rows_v.at[sl], sems.at[sl])

    def w_th(j, sl):
        return pltpu.make_async_copy(
            rows_v.at[sl], out_hbm.at[c, pl.ds(row0 + j * K, K)],
            sems.at[2 + sl])

    n_wo = ROWS_PT // K
    w_st(0, 0).start()
    for j in range(n_wo):
        sl = j % 2
        w_st(j, sl).wait()
        if j + 1 < n_wo:
            if j >= 1:
                w_th(j - 1, (j - 1) % 2).wait()
            w_st(j + 1, (j + 1) % 2).start()
        w_th(j, sl).start()
    w_th(n_wo - 2, (n_wo - 2) % 2).wait()
    w_th(n_wo - 1, (n_wo - 1) % 2).wait()


BM = 2048


def _tc_mm_body(x_ref, w_ref, h_ref):
    h_ref[...] = jnp.dot(x_ref[...], w_ref[...],
                         preferred_element_type=jnp.float32)


def _tc_mm(x_p, w1):
    return pl.pallas_call(
        _tc_mm_body,
        grid=(N_PAD // BM,),
        in_specs=[
            pl.BlockSpec((BM, D), lambda i: (i, 0)),
            pl.BlockSpec((D, D), lambda i: (0, 0)),
        ],
        out_specs=pl.BlockSpec((BM, D), lambda i: (i, 0)),
        out_shape=jax.ShapeDtypeStruct((N_PAD, D), jnp.float32),
    )(x_p, w1)


def _tc1_body(h_ref, degp_ref, hp_ref, dinv_ref):
    deg = degp_ref[0, :] + degp_ref[1, :] + 1.0
    dinv = (1.0 / jnp.sqrt(deg))[:, None]
    hp_ref[...] = h_ref[...] * dinv
    dinv_ref[...] = dinv


def _tc1(h, degp):
    return pl.pallas_call(
        _tc1_body,
        grid=(N_PAD // BM,),
        in_specs=[
            pl.BlockSpec((BM, D), lambda i: (i, 0)),
            pl.BlockSpec((NC, BM), lambda i: (0, i)),
        ],
        out_specs=[
            pl.BlockSpec((BM, D), lambda i: (i, 0)),
            pl.BlockSpec((BM, 1), lambda i: (i, 0)),
        ],
        out_shape=[
            jax.ShapeDtypeStruct((N_PAD, D), jnp.float32),
            jax.ShapeDtypeStruct((N_PAD, 1), jnp.float32),
        ],
    )(h, degp)


def _tc2_body(part_ref, hp_ref, dinv_ref, b_ref, w_ref, out_ref):
    t = (part_ref[0] + part_ref[1] + hp_ref[...]) * dinv_ref[...] + b_ref[...]
    t = jnp.maximum(t, 0.0)
    h = jnp.dot(t, w_ref[...], preferred_element_type=jnp.float32)
    out_ref[...] = h * dinv_ref[...]


def _tc2(part, hp, dinv, b, w):
    return pl.pallas_call(
        _tc2_body,
        grid=(N_PAD // BM,),
        in_specs=[
            pl.BlockSpec((NC, BM, D), lambda i: (0, i, 0)),
            pl.BlockSpec((BM, D), lambda i: (i, 0)),
            pl.BlockSpec((BM, 1), lambda i: (i, 0)),
            pl.BlockSpec((1, D), lambda i: (0, 0)),
            pl.BlockSpec((D, D), lambda i: (0, 0)),
        ],
        out_specs=pl.BlockSpec((BM, D), lambda i: (i, 0)),
        out_shape=jax.ShapeDtypeStruct((N_PAD, D), jnp.float32),
    )(part, hp, dinv, b.reshape(1, D), w)


def _tc3_body(part_ref, hp_ref, dinv_ref, b_ref, out_ref):
    out_ref[...] = ((part_ref[0] + part_ref[1] + hp_ref[...]) * dinv_ref[...]
                    + b_ref[...])


def _tc3(part, hp, dinv, b):
    bm = 1000
    return pl.pallas_call(
        _tc3_body,
        grid=(N // bm,),
        in_specs=[
            pl.BlockSpec((NC, bm, D), lambda i: (0, i, 0)),
            pl.BlockSpec((bm, D), lambda i: (i, 0)),
            pl.BlockSpec((bm, 1), lambda i: (i, 0)),
            pl.BlockSpec((1, D), lambda i: (0, 0)),
        ],
        out_specs=pl.BlockSpec((bm, D), lambda i: (i, 0)),
        out_shape=jax.ShapeDtypeStruct((N, D), jnp.float32),
    )(part, hp, dinv, b.reshape(1, D))


def _tc_ques_body(q_ref, w_ref, b_ref, out_ref):
    out_ref[...] = jnp.dot(q_ref[...], w_ref[...],
                           preferred_element_type=jnp.float32) + b_ref[...]


def _tc_ques(q_emb, wq, bq):
    return pl.pallas_call(
        _tc_ques_body,
        out_shape=jax.ShapeDtypeStruct(q_emb.shape, jnp.float32),
    )(q_emb, wq, bq.reshape(1, D))


def kernel(x, edge_index, W1, b1, W2, b2, Wq, bq, q_emb):
    dst = edge_index[1]
    fill = N + jnp.arange(FILL, dtype=jnp.int32)
    dst_pad = jnp.concatenate(
        [dst.reshape(NW, EPT_REAL),
         jnp.broadcast_to(fill, (NW, FILL))], axis=1)
    dst_p = dst_pad.reshape(NW, CHUNKS, K)
    dst_p_wide = dst_pad.reshape(NW, CHUNKS // 2, 2 * K)
    x_p = jnp.pad(x, ((0, N_PAD - N), (0, 0)))

    ones_k = jnp.ones((2 * K,), jnp.float32)
    zeros_k = jnp.zeros((2 * K,), jnp.float32)
    zrows = jnp.zeros((K, D), jnp.float32)

    degp = _sc_degree(dst_p_wide, ones_k, zeros_k)
    h1 = _tc_mm(x_p, W1)
    hp1, dinv = _tc1(h1, degp)
    src_flat = edge_index[0]
    part1 = _sc_scatter(hp1, src_flat, dst_p, fill, zrows)
    hp2 = _tc2(part1, hp1, dinv, b1, W2)
    part2 = _sc_scatter(hp2, src_flat, dst_p, fill, zrows)
    h2 = _tc3(part2, hp2, dinv, b2)
    ques = _tc_ques(q_emb, Wq, bq)
    return (ques, h2)
